# Initial kernel scaffold; baseline (speedup 1.0000x reference)
import jax
import jax.numpy as jnp
from jax.experimental import pallas as pl


def kernel(x, rings):
    raise NotImplementedError("write your pallas kernel here")


# speedup vs baseline: 23.6351x; 23.6351x over previous
import functools

import numpy as np
import jax
import jax.numpy as jnp
from jax import lax
from jax.experimental import pallas as pl
from jax.experimental.pallas import tpu as pltpu
from jax.experimental.pallas import tpu_sc as plsc

NC, NS, L = 2, 16, 16
NW = NC * NS


def _mesh():
    return plsc.VectorSubcoreMesh(
        core_axis_name="c", subcore_axis_name="s", num_cores=NC, num_subcores=NS
    )


def _make_k1(B, HW, NR, NRp):
    @functools.partial(
        pl.kernel,
        out_type=jax.ShapeDtypeStruct((B, HW), jnp.int32),
        mesh=_mesh(),
        compiler_params=pltpu.CompilerParams(needs_layout_passes=False),
        scratch_types=[
            pltpu.VMEM((HW,), jnp.int32),
            pltpu.VMEM((HW,), jnp.int32),
            pltpu.VMEM((NRp,), jnp.int32),
            pltpu.VMEM((NRp,), jnp.int32),
            pltpu.VMEM((NRp,), jnp.int32),
            pltpu.VMEM((NRp,), jnp.int32),
        ],
    )
    def k1(xc0_hbm, pp_hbm, s0_hbm, n_hbm, nch_hbm, g_hbm, buf0, buf1, s0v, nv, nchv, sv):
        wid = lax.axis_index("s") * NC + lax.axis_index("c")

        @pl.when(wid < B)
        def _():
            b = wid
            pltpu.sync_copy(xc0_hbm.at[b], buf0)
            pltpu.sync_copy(pp_hbm, buf1)
            pltpu.sync_copy(s0_hbm, s0v)
            pltpu.sync_copy(n_hbm, nv)
            pltpu.sync_copy(nch_hbm, nchv)
            iota = lax.iota(jnp.int32, L)
            BIG = jnp.int32(2**30)

            def ring_body(k, _):
                kk = jnp.full((L,), k, jnp.int32)
                s0 = plsc.load_gather(s0v, [kk])
                n = plsc.load_gather(nv, [kk])
                nch_s = jnp.max(plsc.load_gather(nchv, [kk]))
                end = s0 + n

                def chunk_body(c, carry):
                    bv, bi = carry
                    jidx = s0 + c * 16 + iota
                    m = jidx < end
                    jc = jnp.minimum(jidx, jnp.int32(HW - 1))
                    pv = plsc.load_gather(buf1, [jc])
                    pidx = pv & 0xFFFF
                    v = plsc.bitcast(plsc.load_gather(buf0, [pidx]), jnp.float32)
                    v = jnp.where(m, v, -jnp.inf)
                    ji = jnp.where(m, jidx, BIG)
                    better = (v > bv) | ((v == bv) & (ji < bi))
                    bv = jnp.where(better, v, bv)
                    bi = jnp.where(better, ji, bi)
                    return bv, bi

                bv0 = jnp.full((L,), -jnp.inf, jnp.float32)
                bi0 = jnp.full((L,), BIG, jnp.int32)
                bv, bi = lax.fori_loop(0, nch_s, chunk_body, (bv0, bi0))
                mv = jnp.max(bv)
                cand = jnp.where(bv == jnp.full((L,), mv, jnp.float32), bi, BIG)
                argj = jnp.min(cand)
                sval = argj - jnp.max(s0)
                plsc.store_scatter(sv, [kk], jnp.full((L,), sval, jnp.int32), mask=iota == 0)
                return 0

            lax.fori_loop(0, NR, ring_body, 0)

            def b_body(c, _):
                jidx = c * 16 + iota
                pv = buf1[pl.ds(c * 16, 16)]
                sid = lax.shift_right_logical(pv, 16)
                dst = pv & 0xFFFF
                s0 = plsc.load_gather(s0v, [sid])
                n = plsc.load_gather(nv, [sid])
                s = plsc.load_gather(sv, [sid])
                off = jidx - s0 + s
                off = jnp.where(off >= n, off - n, off)
                src = plsc.load_gather(buf1, [s0 + off]) & 0xFFFF
                plsc.store_scatter(buf0, [dst], src)
                return 0

            lax.fori_loop(0, HW // 16, b_body, 0)
            pltpu.sync_copy(buf0, g_hbm.at[b])

    return k1


def _make_k2(P, HW, B, CH):
    PW = P // NW
    NCHK = HW // CH
    assert PW % 2 == 0

    @functools.partial(
        pl.kernel,
        out_type=jax.ShapeDtypeStruct((P * NCHK, CH), jnp.float32),
        mesh=_mesh(),
        compiler_params=pltpu.CompilerParams(needs_layout_passes=False),
        scratch_types=[
            pltpu.VMEM((HW,), jnp.int32),
            pltpu.VMEM((HW,), jnp.float32),
            pltpu.VMEM((CH,), jnp.float32),
            pltpu.VMEM((CH,), jnp.float32),
            pltpu.SemaphoreType.DMA,
            pltpu.SemaphoreType.DMA,
        ],
    )
    def k2(x_hbm, g_hbm, out_hbm, gbuf, xbuf, obuf0, obuf1, sem0, sem1):
        wid = lax.axis_index("s") * NC + lax.axis_index("c")
        b = wid // (NW // B)
        pltpu.sync_copy(g_hbm.at[b], gbuf)
        sems = (sem0, sem1)
        obufs = (obuf0, obuf1)

        def drain(slot):
            pltpu.make_async_copy(
                out_hbm.at[wid * PW * NCHK], obufs[slot], sems[slot]
            ).wait()

        def super_body(sp, _):
            for pp in range(2):
                p = sp * 2 + pp
                i = wid * PW + p
                pltpu.sync_copy(x_hbm.at[i], xbuf)
                for o in range(NCHK):
                    q = pp * NCHK + o
                    slot = q % 2
                    if q >= 2:
                        drain(slot)
                    else:

                        @pl.when(sp > 0)
                        def _():
                            drain(slot)

                    ob = obufs[slot]

                    @plsc.parallel_loop(0, CH // 16, unroll=8)
                    def _(ci):
                        idx = gbuf[pl.ds(o * CH + ci * 16, 16)]
                        ob[pl.ds(ci * 16, 16)] = plsc.load_gather(xbuf, [idx])

                    pltpu.async_copy(ob, out_hbm.at[i * NCHK + o], sems[slot])
            return 0

        lax.fori_loop(0, PW // 2, super_body, 0)
        drain(0)
        drain(1)

    return k2


def kernel(x, rings):
    B, C, H, W = x.shape
    HW = H * W
    lens = [int(r.shape[0]) for r in rings]
    NR = len(rings)
    assert sum(lens) == HW, "rings must tile the image exactly"
    assert HW % 16 == 0 and (B * C) % NW == 0

    starts = np.concatenate([[0], np.cumsum(lens)[:-1]]).astype(np.int32)
    nch = np.asarray([(n + 15) // 16 for n in lens], np.int32)
    NRp = 128
    pad = NRp - NR
    s0_t = jnp.asarray(np.pad(starts, (0, pad)))
    n_t = jnp.asarray(np.pad(np.asarray(lens, np.int32), (0, pad), constant_values=1))
    nch_t = jnp.asarray(np.pad(nch, (0, pad)))

    sid = np.repeat(np.arange(NR, dtype=np.int32), lens)
    perm = jnp.concatenate(
        [r[:, 0].astype(jnp.int32) * W + r[:, 1].astype(jnp.int32) for r in rings]
    )
    pp = perm | jnp.asarray(sid << 16)

    xc0 = lax.bitcast_convert_type(x[:, 0].reshape(B, HW), jnp.int32)
    g = _make_k1(B, HW, NR, NRp)(xc0, pp, s0_t, n_t, nch_t)

    CH = 7168
    out = _make_k2(B * C, HW, B, CH)(x.reshape(B * C, HW), g)
    return out.reshape(B, C, H, W)
